# Initial kernel scaffold; baseline (speedup 1.0000x reference)
import jax
import jax.numpy as jnp
from jax.experimental import pallas as pl


def kernel(time, pitch, duration, pitch_hint, W_pitch, W_pos, W_dur, W_beat, freq_table):
    raise NotImplementedError("write your pallas kernel here")


# speedup vs baseline: 8.1286x; 8.1286x over previous
import functools

import jax
import jax.numpy as jnp
from jax import lax
from jax.experimental import pallas as pl
from jax.experimental.pallas import tpu as pltpu
from jax.experimental.pallas import tpu_sc as plsc

_MAX_BEAT = 256
_MAX_DUR = 192
_L = 16
_NC = 2
_NS = 16
_NW = _NC * _NS
_CHUNK = 256
_F = 129

_OFF_POS = 129
_OFF_BEAT = 129 + 24
_OFF_DUR = 129 + 24 + 257
_TALL_ROWS = 129 + 24 + 257 + 193


def _mask_body(p_ref, o_ref):
    o_ref[...] = p_ref[...] != 0


def _sc_body(tall_hbm, freq_hbm, time_hbm, pitch_hbm, dur_hbm, hint_hbm,
             out_hbm, tall_v, freq_v, time_v, pitch_v, dur_v, hint_v, out_v):
    tw = time_hbm.shape[0] // _NW
    n_chunks = tw // _CHUNK
    chunks_per_row = 2048 // _CHUNK
    wid = lax.axis_index("s") * _NC + lax.axis_index("c")

    pltpu.sync_copy(tall_hbm, tall_v)
    pltpu.sync_copy(freq_hbm, freq_v)
    pltpu.sync_copy(hint_hbm.at[pl.ds(wid * 32, 32)], hint_v)
    iota = lax.broadcasted_iota(jnp.int32, (_L,), 0)

    def chunk_body(g, _):
        base = wid * tw + g * _CHUNK
        pltpu.sync_copy(time_hbm.at[pl.ds(base, _CHUNK)], time_v)
        pltpu.sync_copy(pitch_hbm.at[pl.ds(base, _CHUNK)], pitch_v)
        pltpu.sync_copy(dur_hbm.at[pl.ds(base, _CHUNK)], dur_v)

        r = g // chunks_per_row
        phv = plsc.load_gather(hint_v, [4 * r + jnp.minimum(iota, 3)])
        h = [plsc.load_gather(tall_v, [phv[j] * 16 + iota])
             for j in range(4)]

        def grp_body(gi, carry):
            tv = time_v[pl.ds(gi * _L, _L)]
            pv = pitch_v[pl.ds(gi * _L, _L)]
            dv = dur_v[pl.ds(gi * _L, _L)]
            qv = (tv * 43691) >> 20
            a0 = pv
            a1 = _OFF_POS + (tv - qv * 24)
            a2 = _OFF_BEAT + jnp.minimum(qv, _MAX_BEAT)
            a3 = _OFF_DUR + jnp.minimum(dv, _MAX_DUR)
            avs = (a0, a1, a2, a3)
            base16 = gi * _L * _F
            for j in range(_L):
                o = base16 + j * _F
                for k in range(4):
                    row = plsc.load_gather(tall_v, [avs[k][j] * 16 + iota])
                    out_v[pl.ds(o + k * 16, _L)] = row
                for k in range(4):
                    out_v[pl.ds(o + 65 + k * 16, _L)] = h[k]
            return carry

        lax.fori_loop(0, _CHUNK // _L, grp_body, 0)

        def frq_body(j, carry):
            pv = pitch_v[pl.ds(j * _L, _L)]
            fv = plsc.load_gather(freq_v, [pv])
            plsc.store_scatter(out_v, [(iota + j * _L) * _F + 64], fv)
            return carry

        lax.fori_loop(0, _CHUNK // _L, frq_body, 0)

        pltpu.sync_copy(out_v, out_hbm.at[pl.ds(base * _F, _CHUNK * _F)])
        return _

    lax.fori_loop(0, n_chunks, chunk_body, 0)


def kernel(time, pitch, duration, pitch_hint, W_pitch, W_pos, W_dur, W_beat,
           freq_table):
    B, S = time.shape
    T = B * S
    tall = jnp.concatenate([W_pitch, W_pos, W_beat, W_dur], axis=0).reshape(-1)
    freq = jnp.pad(freq_table.reshape(-1), (0, 7))

    mesh = plsc.VectorSubcoreMesh(core_axis_name="c", subcore_axis_name="s",
                                  num_cores=_NC, num_subcores=_NS)
    sc = pl.kernel(
        _sc_body,
        out_type=jax.ShapeDtypeStruct((T * _F,), jnp.float32),
        mesh=mesh,
        compiler_params=pltpu.CompilerParams(needs_layout_passes=False),
        scratch_types=[
            pltpu.VMEM((_TALL_ROWS * 16,), jnp.float32),
            pltpu.VMEM((136,), jnp.float32),
            pltpu.VMEM((_CHUNK,), jnp.int32),
            pltpu.VMEM((_CHUNK,), jnp.int32),
            pltpu.VMEM((_CHUNK,), jnp.int32),
            pltpu.VMEM((32,), jnp.int32),
            pltpu.VMEM((_CHUNK * _F,), jnp.float32),
        ],
    )
    out_flat = sc(tall, freq, time.reshape(-1), pitch.reshape(-1),
                  duration.reshape(-1), pitch_hint.reshape(-1))
    tensor_out = out_flat.reshape(B, S, _F)

    mask = pl.pallas_call(
        _mask_body,
        out_shape=jax.ShapeDtypeStruct((B, S), jnp.bool_),
        grid=(B // 8,),
        in_specs=[pl.BlockSpec((8, S), lambda i: (i, 0))],
        out_specs=pl.BlockSpec((8, S), lambda i: (i, 0)),
    )(pitch)
    return tensor_out, mask
